# baseline (speedup 1.0000x reference)
import functools

import jax
import jax.numpy as jnp
from jax import lax
from jax.experimental import pallas as pl
from jax.experimental.pallas import tpu as pltpu
from jax.experimental.pallas import tpu_sc as plsc

N = 10000
E = 320000
D = 128
H = 128
G = 128

NC = 2
NS = 16
LB = 128
K = 160
GR = 8
EP = NS * K * LB
LBP = 64
KP = EP // (NS * LBP)
GRP = 16
ND = 5
NP = 10240
RPS = NP // NS
R = 512
NB = NP // R

_mesh = plsc.VectorSubcoreMesh(
    core_axis_name="c", subcore_axis_name="s", num_cores=NC, num_subcores=NS)


def _deg_body(dst_all, ones_hbm, z_hbm, deg_out, dst_v, ones_v, sh_deg,
              sem_a, sem_b):
  cid = lax.axis_index("c")
  sid = lax.axis_index("s")
  pltpu.sync_copy(dst_all.at[cid, sid], dst_v)
  pltpu.sync_copy(ones_hbm, ones_v)
  pltpu.sync_copy(z_hbm, sh_deg.at[pl.ds(sid * RPS, RPS)])
  plsc.subcore_barrier()

  def body(g, carry):
    ds = []
    for j in range(16):
      sem = sem_a if j < 8 else sem_b
      ds.append(pltpu.async_copy(ones_v, sh_deg.at[dst_v.at[g * 16 + j]],
                                 sem, add=True))
    for d in ds:
      d.wait()
    return carry

  lax.fori_loop(0, K // 16, body, 0)
  plsc.subcore_barrier()
  pltpu.sync_copy(sh_deg.at[pl.ds(sid * RPS, RPS)],
                  deg_out.at[cid, pl.ds(sid * RPS, RPS)])


_deg_kernel = pl.kernel(
    _deg_body,
    out_type=jax.ShapeDtypeStruct((NC, NP, 16), jnp.float32),
    mesh=_mesh,
    scratch_types=[
        pltpu.VMEM((K, LB), jnp.int32),
        pltpu.VMEM((LB, 16), jnp.float32),
        pltpu.VMEM_SHARED((NP, 16), jnp.float32),
        pltpu.SemaphoreType.DMA,
        pltpu.SemaphoreType.DMA,
    ],
)


def _prop_body(src_all, dst_all, y_o, y_c, z_hbm, acc_o, acc_c,
               src_v, dst_v, r0, r1, r2, r3, r4,
               sh_acc, g0, g1, g2, g3, g4, s0, s1, s2, s3, s4):
  cid = lax.axis_index("c")
  sid = lax.axis_index("s")
  pltpu.sync_copy(z_hbm, sh_acc.at[pl.ds(sid * RPS, RPS)])
  plsc.subcore_barrier()

  rows = (r0, r1, r2, r3, r4)
  gsem = (g0, g1, g2, g3, g4)
  ssem = (s0, s1, s2, s3, s4)

  def run_branch(y_ref, acc_ref):
    def body(g, carry):
      pltpu.sync_copy(src_all.at[cid, sid, pl.ds(g * GRP, GRP)], src_v)
      pltpu.sync_copy(dst_all.at[cid, sid, pl.ds(g * GRP, GRP)], dst_v)
      gd = [None] * ND
      sd = [None] * ND
      for j in range(ND - 1):
        gd[j] = pltpu.async_copy(y_ref.at[src_v.at[j]], rows[j], gsem[j])
      for j in range(GRP):
        p = j % ND
        gd[p].wait()
        nj = j + ND - 1
        if nj < GRP:
          q = nj % ND
          if sd[q] is not None:
            sd[q].wait()
          gd[q] = pltpu.async_copy(y_ref.at[src_v.at[nj]], rows[q], gsem[q])
        sd[p] = pltpu.async_copy(rows[p], sh_acc.at[dst_v.at[j]], ssem[p],
                                 add=True)
      for j in range(GRP - ND, GRP):
        sd[j % ND].wait()
      return carry

    lax.fori_loop(0, KP // GRP, body, 0)
    plsc.subcore_barrier()
    pltpu.sync_copy(sh_acc.at[pl.ds(sid * RPS, RPS)],
                    acc_ref.at[pl.ds(sid * RPS, RPS)])

  @pl.when(cid == 0)
  def _():
    run_branch(y_o, acc_o)

  @pl.when(cid == 1)
  def _():
    run_branch(y_c, acc_c)


_prop_kernel = pl.kernel(
    _prop_body,
    out_type=(jax.ShapeDtypeStruct((NP, H), jnp.float32),
              jax.ShapeDtypeStruct((NP, H), jnp.float32)),
    mesh=_mesh,
    scratch_types=[
        pltpu.VMEM((GRP, LBP), jnp.int32),
        pltpu.VMEM((GRP, LBP), jnp.int32),
        pltpu.VMEM((LBP, H), jnp.float32),
        pltpu.VMEM((LBP, H), jnp.float32),
        pltpu.VMEM((LBP, H), jnp.float32),
        pltpu.VMEM((LBP, H), jnp.float32),
        pltpu.VMEM((LBP, H), jnp.float32),
        pltpu.VMEM_SHARED((NP, H), jnp.float32),
    ] + [pltpu.SemaphoreType.DMA] * 10,
)


def _xw_body(x_o_ref, x_c_ref, w1o_ref, w1c_ref, xw_o_ref, xw_c_ref):
  xw_o_ref[...] = jnp.dot(x_o_ref[...], w1o_ref[...],
                          preferred_element_type=jnp.float32)
  xw_c_ref[...] = jnp.dot(x_c_ref[...], w1c_ref[...],
                          preferred_element_type=jnp.float32)


def _pre_body(xw_o_ref, xw_c_ref, deg_ref, y_o_ref, y_c_ref):
  dis_o = lax.rsqrt(deg_ref[0, :, 0:1] + 1.0)
  dis_c = lax.rsqrt(deg_ref[1, :, 0:1] + 1.0)
  y_o_ref[...] = xw_o_ref[...] * dis_o
  y_c_ref[...] = xw_c_ref[...] * dis_c


def _mid_body(acc_o_ref, acc_c_ref, y_o_ref, y_c_ref, deg_ref,
              b1o_ref, b1c_ref, w2o_ref, w2c_ref, y2o_ref, y2c_ref):
  dis_o = lax.rsqrt(deg_ref[0, :, 0:1] + 1.0)
  dis_c = lax.rsqrt(deg_ref[1, :, 0:1] + 1.0)
  h_o = jnp.maximum(dis_o * (acc_o_ref[...] + y_o_ref[...]) + b1o_ref[...],
                    0.0)
  h_c = jnp.maximum(dis_c * (acc_c_ref[...] + y_c_ref[...]) + b1c_ref[...],
                    0.0)
  y2o_ref[...] = jnp.dot(h_o, w2o_ref[...],
                         preferred_element_type=jnp.float32) * dis_o
  y2c_ref[...] = jnp.dot(h_c, w2c_ref[...],
                         preferred_element_type=jnp.float32) * dis_c


def _fin_body(acc_o_ref, acc_c_ref, y2o_ref, y2c_ref, deg_ref,
              b2o_ref, b2c_ref, batch_ref, wl_ref, bl_ref,
              x2o_ref, x2c_ref, hout_ref, pool_acc):
  i = pl.program_id(0)
  dis_o = lax.rsqrt(deg_ref[0, :, 0:1] + 1.0)
  dis_c = lax.rsqrt(deg_ref[1, :, 0:1] + 1.0)
  x2o = dis_o * (acc_o_ref[...] + y2o_ref[...]) + b2o_ref[...]
  x2c = dis_c * (acc_c_ref[...] + y2c_ref[...]) + b2c_ref[...]
  x2o_ref[...] = x2o
  x2c_ref[...] = x2c
  gi = lax.broadcasted_iota(jnp.int32, (G, R), 0).astype(jnp.float32)
  mt = jnp.where(batch_ref[0] == gi, 1.0, 0.0)
  contrib = jnp.dot(mt, x2c, preferred_element_type=jnp.float32)

  @pl.when(i == 0)
  def _():
    pool_acc[...] = contrib

  @pl.when(i > 0)
  def _():
    pool_acc[...] = pool_acc[...] + contrib

  @pl.when(i == NB - 1)
  def _():
    hout_ref[...] = jnp.dot(pool_acc[...], wl_ref[...],
                            preferred_element_type=jnp.float32) + bl_ref[...]


def _row_spec():
  return pl.BlockSpec((R, H), lambda i: (i, 0))


def _deg_spec():
  return pl.BlockSpec((NC, R, 16), lambda i: (0, i, 0))


def _full_spec():
  return pl.BlockSpec((D, H), lambda i: (0, 0))


def _bias_spec():
  return pl.BlockSpec((1, H), lambda i: (0, 0))


_xw_call = pl.pallas_call(
    _xw_body,
    grid=(NB,),
    in_specs=[_row_spec(), _row_spec(), _full_spec(), _full_spec()],
    out_specs=(_row_spec(), _row_spec()),
    out_shape=(jax.ShapeDtypeStruct((NP, H), jnp.float32),
               jax.ShapeDtypeStruct((NP, H), jnp.float32)),
)

_pre_call = pl.pallas_call(
    _pre_body,
    grid=(NB,),
    in_specs=[_row_spec(), _row_spec(), _deg_spec()],
    out_specs=(_row_spec(), _row_spec()),
    out_shape=(jax.ShapeDtypeStruct((NP, H), jnp.float32),
               jax.ShapeDtypeStruct((NP, H), jnp.float32)),
)

_mid_call = pl.pallas_call(
    _mid_body,
    grid=(NB,),
    in_specs=[_row_spec(), _row_spec(), _row_spec(), _row_spec(), _deg_spec(),
              _bias_spec(), _bias_spec(), _full_spec(), _full_spec()],
    out_specs=(_row_spec(), _row_spec()),
    out_shape=(jax.ShapeDtypeStruct((NP, H), jnp.float32),
               jax.ShapeDtypeStruct((NP, H), jnp.float32)),
)

_fin_call = pl.pallas_call(
    _fin_body,
    grid=(NB,),
    in_specs=[_row_spec(), _row_spec(), _row_spec(), _row_spec(), _deg_spec(),
              _bias_spec(), _bias_spec(),
              pl.BlockSpec((1, 1, R), lambda i: (i, 0, 0)),
              _full_spec(), _bias_spec()],
    out_specs=(_row_spec(), _row_spec(),
               pl.BlockSpec((G, H), lambda i: (0, 0))),
    out_shape=(jax.ShapeDtypeStruct((N, H), jnp.float32),
               jax.ShapeDtypeStruct((N, H), jnp.float32),
               jax.ShapeDtypeStruct((G, H), jnp.float32)),
    scratch_shapes=[pltpu.VMEM((G, H), jnp.float32)],
)


def _pad_edges(ei):
  src = ei[0].astype(jnp.int32)
  dst = ei[1].astype(jnp.int32)
  pad = jnp.full((EP - E,), N, dtype=jnp.int32)
  src = jnp.concatenate([src, pad]).reshape(NS, K, LB)
  dst = jnp.concatenate([dst, pad]).reshape(NS, K, LB)
  return src, dst


@jax.jit
def kernel(x_o, x_c, edge_index_o, edge_index_c, batch_o,
           W1o, b1o, W2o, b2o, W1c, b1c, W2c, b2c, Wl1, bl1):
  f32 = jnp.float32
  src_o, dst_o = _pad_edges(edge_index_o)
  src_c, dst_c = _pad_edges(edge_index_c)
  src_all = jnp.stack([src_o, src_c])
  dst_all = jnp.stack([dst_o, dst_c])
  src_all_p = src_all.reshape(NC, NS, KP, LBP)
  dst_all_p = dst_all.reshape(NC, NS, KP, LBP)

  xpad = jnp.zeros((NP - N, D), f32)
  x_o_p = jnp.concatenate([x_o.astype(f32), xpad])
  x_c_p = jnp.concatenate([x_c.astype(f32), xpad])

  ones_hbm = jnp.concatenate(
      [jnp.ones((LB, 1), f32), jnp.zeros((LB, 15), f32)], axis=1)
  z16 = jnp.zeros((RPS, 16), f32)
  z128 = jnp.zeros((RPS, H), f32)

  deg_all = _deg_kernel(dst_all, ones_hbm, z16)
  xw_o, xw_c = _xw_call(x_o_p, x_c_p, W1o.astype(f32), W1c.astype(f32))

  y1o, y1c = _pre_call(xw_o, xw_c, deg_all)
  acc1o, acc1c = _prop_kernel(src_all_p, dst_all_p, y1o, y1c, z128)
  y2o, y2c = _mid_call(acc1o, acc1c, y1o, y1c, deg_all,
                       b1o.reshape(1, H).astype(f32),
                       b1c.reshape(1, H).astype(f32),
                       W2o.astype(f32), W2c.astype(f32))
  acc2o, acc2c = _prop_kernel(src_all_p, dst_all_p, y2o, y2c, z128)

  batch_p = jnp.concatenate(
      [batch_o.astype(jnp.int32), jnp.full((NP - N,), G, jnp.int32)])
  batch_f = batch_p.astype(f32).reshape(NB, 1, R)

  x2o, x2c, hout = _fin_call(acc2o, acc2c, y2o, y2c, deg_all,
                             b2o.reshape(1, H).astype(f32),
                             b2c.reshape(1, H).astype(f32),
                             batch_f, Wl1.astype(f32),
                             bl1.reshape(1, H).astype(f32))
  return (hout, x2o, x2c)


# speedup vs baseline: 1.0597x; 1.0597x over previous
import functools

import jax
import jax.numpy as jnp
from jax import lax
from jax.experimental import pallas as pl
from jax.experimental.pallas import tpu as pltpu
from jax.experimental.pallas import tpu_sc as plsc

N = 10000
E = 320000
D = 128
H = 128
G = 128

NC = 2
NS = 16
LB = 128
K = 160
GR = 8
EP = NS * K * LB
LBP = 64
KP = EP // (NS * LBP)
GRP = 16
ND = 5
NP = 10240
RPS = NP // NS
R = 512
NB = NP // R

_mesh = plsc.VectorSubcoreMesh(
    core_axis_name="c", subcore_axis_name="s", num_cores=NC, num_subcores=NS)


def _deg_body(dst_all, ones_hbm, z_hbm, deg_out, dst_v, ones_v, sh_deg,
              sem_a, sem_b):
  cid = lax.axis_index("c")
  sid = lax.axis_index("s")
  pltpu.sync_copy(dst_all.at[cid, sid], dst_v)
  pltpu.sync_copy(ones_hbm, ones_v)
  pltpu.sync_copy(z_hbm, sh_deg.at[pl.ds(sid * RPS, RPS)])
  plsc.subcore_barrier()

  def body(g, carry):
    ds = []
    for j in range(16):
      sem = sem_a if j < 8 else sem_b
      ds.append(pltpu.async_copy(ones_v, sh_deg.at[dst_v.at[g * 16 + j]],
                                 sem, add=True))
    for d in ds:
      d.wait()
    return carry

  lax.fori_loop(0, K // 16, body, 0)
  plsc.subcore_barrier()
  pltpu.sync_copy(sh_deg.at[pl.ds(sid * RPS, RPS)],
                  deg_out.at[cid, pl.ds(sid * RPS, RPS)])


_deg_kernel = pl.kernel(
    _deg_body,
    out_type=jax.ShapeDtypeStruct((NC, NP, 16), jnp.float32),
    mesh=_mesh,
    scratch_types=[
        pltpu.VMEM((K, LB), jnp.int32),
        pltpu.VMEM((LB, 16), jnp.float32),
        pltpu.VMEM_SHARED((NP, 16), jnp.float32),
        pltpu.SemaphoreType.DMA,
        pltpu.SemaphoreType.DMA,
    ],
)


def _prop_body(src_all, dst_all, y_o, y_c, z_hbm, acc_o, acc_c,
               src_v, dst_v, r0, r1, r2, r3, r4,
               sh_acc, g0, g1, g2, g3, g4, s0, s1, s2, s3, s4):
  cid = lax.axis_index("c")
  sid = lax.axis_index("s")
  pltpu.sync_copy(z_hbm, sh_acc.at[pl.ds(sid * RPS, RPS)])
  plsc.subcore_barrier()

  rows = (r0, r1, r2, r3, r4)
  gsem = (g0, g1, g2, g3, g4)
  ssem = (s0, s1, s2, s3, s4)

  def run_branch(y_ref, acc_ref):
    def body(g, carry):
      pltpu.sync_copy(src_all.at[cid, sid, pl.ds(g * GRP, GRP)], src_v)
      pltpu.sync_copy(dst_all.at[cid, sid, pl.ds(g * GRP, GRP)], dst_v)
      gd = [None] * ND
      sd = [None] * ND
      for j in range(ND - 1):
        gd[j] = pltpu.async_copy(y_ref.at[src_v.at[j]], rows[j], gsem[j])
      for j in range(GRP):
        p = j % ND
        gd[p].wait()
        nj = j + ND - 1
        if nj < GRP:
          q = nj % ND
          if sd[q] is not None:
            sd[q].wait()
          gd[q] = pltpu.async_copy(y_ref.at[src_v.at[nj]], rows[q], gsem[q])
        sd[p] = pltpu.async_copy(rows[p], sh_acc.at[dst_v.at[j]], ssem[p],
                                 add=True)
      for j in range(GRP - ND, GRP):
        sd[j % ND].wait()
      return carry

    lax.fori_loop(0, KP // GRP, body, 0)
    plsc.subcore_barrier()
    pltpu.sync_copy(sh_acc.at[pl.ds(sid * RPS, RPS)],
                    acc_ref.at[pl.ds(sid * RPS, RPS)])

  @pl.when(cid == 0)
  def _():
    run_branch(y_o, acc_o)

  @pl.when(cid == 1)
  def _():
    run_branch(y_c, acc_c)


_prop_kernel = pl.kernel(
    _prop_body,
    out_type=(jax.ShapeDtypeStruct((NP, H), jnp.float32),
              jax.ShapeDtypeStruct((NP, H), jnp.float32)),
    mesh=_mesh,
    scratch_types=[
        pltpu.VMEM((GRP, LBP), jnp.int32),
        pltpu.VMEM((GRP, LBP), jnp.int32),
        pltpu.VMEM((LBP, H), jnp.float32),
        pltpu.VMEM((LBP, H), jnp.float32),
        pltpu.VMEM((LBP, H), jnp.float32),
        pltpu.VMEM((LBP, H), jnp.float32),
        pltpu.VMEM((LBP, H), jnp.float32),
        pltpu.VMEM_SHARED((NP, H), jnp.float32),
    ] + [pltpu.SemaphoreType.DMA] * 10,
)


def _xw_body(x_o_ref, x_c_ref, w1o_ref, w1c_ref, xw_o_ref, xw_c_ref):
  xw_o_ref[...] = jnp.dot(x_o_ref[...], w1o_ref[...],
                          preferred_element_type=jnp.float32)
  xw_c_ref[...] = jnp.dot(x_c_ref[...], w1c_ref[...],
                          preferred_element_type=jnp.float32)


def _pre_body(xw_o_ref, xw_c_ref, deg_ref, y_o_ref, y_c_ref):
  dis_o = lax.rsqrt(deg_ref[0, :, 0:1] + 1.0)
  dis_c = lax.rsqrt(deg_ref[1, :, 0:1] + 1.0)
  y_o_ref[...] = xw_o_ref[...] * dis_o
  y_c_ref[...] = xw_c_ref[...] * dis_c


def _mid_body(acc_o_ref, acc_c_ref, y_o_ref, y_c_ref, deg_ref,
              b1o_ref, b1c_ref, w2o_ref, w2c_ref, y2o_ref, y2c_ref):
  dis_o = lax.rsqrt(deg_ref[0, :, 0:1] + 1.0)
  dis_c = lax.rsqrt(deg_ref[1, :, 0:1] + 1.0)
  h_o = jnp.maximum(dis_o * (acc_o_ref[...] + y_o_ref[...]) + b1o_ref[...],
                    0.0)
  h_c = jnp.maximum(dis_c * (acc_c_ref[...] + y_c_ref[...]) + b1c_ref[...],
                    0.0)
  y2o_ref[...] = jnp.dot(h_o, w2o_ref[...],
                         preferred_element_type=jnp.float32) * dis_o
  y2c_ref[...] = jnp.dot(h_c, w2c_ref[...],
                         preferred_element_type=jnp.float32) * dis_c


def _fin_body(acc_o_ref, acc_c_ref, y2o_ref, y2c_ref, deg_ref,
              b2o_ref, b2c_ref, batch_ref, wl_ref, bl_ref,
              x2o_ref, x2c_ref, hout_ref, pool_acc):
  i = pl.program_id(0)
  dis_o = lax.rsqrt(deg_ref[0, :, 0:1] + 1.0)
  dis_c = lax.rsqrt(deg_ref[1, :, 0:1] + 1.0)
  x2o = dis_o * (acc_o_ref[...] + y2o_ref[...]) + b2o_ref[...]
  x2c = dis_c * (acc_c_ref[...] + y2c_ref[...]) + b2c_ref[...]
  x2o_ref[...] = x2o
  x2c_ref[...] = x2c
  gi = lax.broadcasted_iota(jnp.int32, (G, R), 0).astype(jnp.float32)
  mt = jnp.where(batch_ref[0] == gi, 1.0, 0.0)
  contrib = jnp.dot(mt, x2c, preferred_element_type=jnp.float32)

  @pl.when(i == 0)
  def _():
    pool_acc[...] = contrib

  @pl.when(i > 0)
  def _():
    pool_acc[...] = pool_acc[...] + contrib

  @pl.when(i == NB - 1)
  def _():
    hout_ref[...] = jnp.dot(pool_acc[...], wl_ref[...],
                            preferred_element_type=jnp.float32) + bl_ref[...]


def _row_spec():
  return pl.BlockSpec((R, H), lambda i: (i, 0))


def _deg_spec():
  return pl.BlockSpec((NC, R, 16), lambda i: (0, i, 0))


def _full_spec():
  return pl.BlockSpec((D, H), lambda i: (0, 0))


def _bias_spec():
  return pl.BlockSpec((1, H), lambda i: (0, 0))


_xw_call = pl.pallas_call(
    _xw_body,
    grid=(NB,),
    in_specs=[_row_spec(), _row_spec(), _full_spec(), _full_spec()],
    out_specs=(_row_spec(), _row_spec()),
    out_shape=(jax.ShapeDtypeStruct((NP, H), jnp.float32),
               jax.ShapeDtypeStruct((NP, H), jnp.float32)),
)

_pre_call = pl.pallas_call(
    _pre_body,
    grid=(NB,),
    in_specs=[_row_spec(), _row_spec(), _deg_spec()],
    out_specs=(_row_spec(), _row_spec()),
    out_shape=(jax.ShapeDtypeStruct((NP, H), jnp.float32),
               jax.ShapeDtypeStruct((NP, H), jnp.float32)),
)

_mid_call = pl.pallas_call(
    _mid_body,
    grid=(NB,),
    in_specs=[_row_spec(), _row_spec(), _row_spec(), _row_spec(), _deg_spec(),
              _bias_spec(), _bias_spec(), _full_spec(), _full_spec()],
    out_specs=(_row_spec(), _row_spec()),
    out_shape=(jax.ShapeDtypeStruct((NP, H), jnp.float32),
               jax.ShapeDtypeStruct((NP, H), jnp.float32)),
)

_fin_call = pl.pallas_call(
    _fin_body,
    grid=(NB,),
    in_specs=[_row_spec(), _row_spec(), _row_spec(), _row_spec(), _deg_spec(),
              _bias_spec(), _bias_spec(),
              pl.BlockSpec((1, 1, R), lambda i: (i, 0, 0)),
              _full_spec(), _bias_spec()],
    out_specs=(_row_spec(), _row_spec(),
               pl.BlockSpec((G, H), lambda i: (0, 0))),
    out_shape=(jax.ShapeDtypeStruct((NP, H), jnp.float32),
               jax.ShapeDtypeStruct((NP, H), jnp.float32),
               jax.ShapeDtypeStruct((G, H), jnp.float32)),
    scratch_shapes=[pltpu.VMEM((G, H), jnp.float32)],
)


def _pad_edges(ei):
  src = ei[0].astype(jnp.int32)
  dst = ei[1].astype(jnp.int32)
  pad = jnp.full((EP - E,), N, dtype=jnp.int32)
  src = jnp.concatenate([src, pad]).reshape(NS, K, LB)
  dst = jnp.concatenate([dst, pad]).reshape(NS, K, LB)
  return src, dst


@jax.jit
def kernel(x_o, x_c, edge_index_o, edge_index_c, batch_o,
           W1o, b1o, W2o, b2o, W1c, b1c, W2c, b2c, Wl1, bl1):
  f32 = jnp.float32
  src_o, dst_o = _pad_edges(edge_index_o)
  src_c, dst_c = _pad_edges(edge_index_c)
  src_all = jnp.stack([src_o, src_c])
  dst_all = jnp.stack([dst_o, dst_c])
  src_all_p = src_all.reshape(NC, NS, KP, LBP)
  dst_all_p = dst_all.reshape(NC, NS, KP, LBP)

  xpad = jnp.zeros((NP - N, D), f32)
  x_o_p = jnp.concatenate([x_o.astype(f32), xpad])
  x_c_p = jnp.concatenate([x_c.astype(f32), xpad])

  ones_hbm = jnp.concatenate(
      [jnp.ones((LB, 1), f32), jnp.zeros((LB, 15), f32)], axis=1)
  z16 = jnp.zeros((RPS, 16), f32)
  z128 = jnp.zeros((RPS, H), f32)

  deg_all = _deg_kernel(dst_all, ones_hbm, z16)
  xw_o, xw_c = _xw_call(x_o_p, x_c_p, W1o.astype(f32), W1c.astype(f32))

  y1o, y1c = _pre_call(xw_o, xw_c, deg_all)
  acc1o, acc1c = _prop_kernel(src_all_p, dst_all_p, y1o, y1c, z128)
  y2o, y2c = _mid_call(acc1o, acc1c, y1o, y1c, deg_all,
                       b1o.reshape(1, H).astype(f32),
                       b1c.reshape(1, H).astype(f32),
                       W2o.astype(f32), W2c.astype(f32))
  acc2o, acc2c = _prop_kernel(src_all_p, dst_all_p, y2o, y2c, z128)

  batch_p = jnp.concatenate(
      [batch_o.astype(jnp.int32), jnp.full((NP - N,), G, jnp.int32)])
  batch_f = batch_p.astype(f32).reshape(NB, 1, R)

  x2o, x2c, hout = _fin_call(acc2o, acc2c, y2o, y2c, deg_all,
                             b2o.reshape(1, H).astype(f32),
                             b2c.reshape(1, H).astype(f32),
                             batch_f, Wl1.astype(f32),
                             bl1.reshape(1, H).astype(f32))
  return (hout, x2o[:N], x2c[:N])


# speedup vs baseline: 1.0829x; 1.0219x over previous
import functools

import jax
import jax.numpy as jnp
from jax import lax
from jax.experimental import pallas as pl
from jax.experimental.pallas import tpu as pltpu
from jax.experimental.pallas import tpu_sc as plsc

N = 10000
E = 320000
D = 128
H = 128
G = 128

NC = 2
NS = 16
LB = 128
K = 160
GR = 8
EP = NS * K * LB
LBP = 64
KP = EP // (NS * LBP)
GRP = 32
ND = 5
NP = 10240
RPS = NP // NS
R = 512
NB = NP // R

_mesh = plsc.VectorSubcoreMesh(
    core_axis_name="c", subcore_axis_name="s", num_cores=NC, num_subcores=NS)


def _deg_body(dst_all, ones_hbm, z_hbm, deg_out, dst_v, ones_v, sh_deg,
              sem_a, sem_b):
  cid = lax.axis_index("c")
  sid = lax.axis_index("s")
  pltpu.sync_copy(dst_all.at[cid, sid], dst_v)
  pltpu.sync_copy(ones_hbm, ones_v)
  pltpu.sync_copy(z_hbm, sh_deg.at[pl.ds(sid * RPS, RPS)])
  plsc.subcore_barrier()

  def body(g, carry):
    ds = []
    for j in range(16):
      sem = sem_a if j < 8 else sem_b
      ds.append(pltpu.async_copy(ones_v, sh_deg.at[dst_v.at[g * 16 + j]],
                                 sem, add=True))
    for d in ds:
      d.wait()
    return carry

  lax.fori_loop(0, K // 16, body, 0)
  plsc.subcore_barrier()
  pltpu.sync_copy(sh_deg.at[pl.ds(sid * RPS, RPS)],
                  deg_out.at[cid, pl.ds(sid * RPS, RPS)])


_deg_kernel = pl.kernel(
    _deg_body,
    out_type=jax.ShapeDtypeStruct((NC, NP, 16), jnp.float32),
    mesh=_mesh,
    scratch_types=[
        pltpu.VMEM((K, LB), jnp.int32),
        pltpu.VMEM((LB, 16), jnp.float32),
        pltpu.VMEM_SHARED((NP, 16), jnp.float32),
        pltpu.SemaphoreType.DMA,
        pltpu.SemaphoreType.DMA,
    ],
)


def _prop_body(src_all, dst_all, y_o, y_c, z_hbm, acc_o, acc_c,
               src_v, dst_v, r0, r1, r2, r3, r4,
               sh_acc, g0, g1, g2, g3, g4, s0, s1, s2, s3, s4):
  cid = lax.axis_index("c")
  sid = lax.axis_index("s")
  pltpu.sync_copy(z_hbm, sh_acc.at[pl.ds(sid * RPS, RPS)])
  plsc.subcore_barrier()

  rows = (r0, r1, r2, r3, r4)
  gsem = (g0, g1, g2, g3, g4)
  ssem = (s0, s1, s2, s3, s4)

  def run_branch(y_ref, acc_ref):
    def body(g, carry):
      pltpu.sync_copy(src_all.at[cid, sid, pl.ds(g * GRP, GRP)], src_v)
      pltpu.sync_copy(dst_all.at[cid, sid, pl.ds(g * GRP, GRP)], dst_v)
      gd = [None] * ND
      sd = [None] * ND
      for j in range(ND - 1):
        gd[j] = pltpu.async_copy(y_ref.at[src_v.at[j]], rows[j], gsem[j])
      for j in range(GRP):
        p = j % ND
        gd[p].wait()
        nj = j + ND - 1
        if nj < GRP:
          q = nj % ND
          if sd[q] is not None:
            sd[q].wait()
          gd[q] = pltpu.async_copy(y_ref.at[src_v.at[nj]], rows[q], gsem[q])
        sd[p] = pltpu.async_copy(rows[p], sh_acc.at[dst_v.at[j]], ssem[p],
                                 add=True)
      for j in range(GRP - ND, GRP):
        sd[j % ND].wait()
      return carry

    lax.fori_loop(0, KP // GRP, body, 0)
    plsc.subcore_barrier()
    pltpu.sync_copy(sh_acc.at[pl.ds(sid * RPS, RPS)],
                    acc_ref.at[pl.ds(sid * RPS, RPS)])

  @pl.when(cid == 0)
  def _():
    run_branch(y_o, acc_o)

  @pl.when(cid == 1)
  def _():
    run_branch(y_c, acc_c)


_prop_kernel = pl.kernel(
    _prop_body,
    out_type=(jax.ShapeDtypeStruct((NP, H), jnp.float32),
              jax.ShapeDtypeStruct((NP, H), jnp.float32)),
    mesh=_mesh,
    scratch_types=[
        pltpu.VMEM((GRP, LBP), jnp.int32),
        pltpu.VMEM((GRP, LBP), jnp.int32),
        pltpu.VMEM((LBP, H), jnp.float32),
        pltpu.VMEM((LBP, H), jnp.float32),
        pltpu.VMEM((LBP, H), jnp.float32),
        pltpu.VMEM((LBP, H), jnp.float32),
        pltpu.VMEM((LBP, H), jnp.float32),
        pltpu.VMEM_SHARED((NP, H), jnp.float32),
    ] + [pltpu.SemaphoreType.DMA] * 10,
)


def _xw_body(x_o_ref, x_c_ref, w1o_ref, w1c_ref, xw_o_ref, xw_c_ref):
  xw_o_ref[...] = jnp.dot(x_o_ref[...], w1o_ref[...],
                          preferred_element_type=jnp.float32)
  xw_c_ref[...] = jnp.dot(x_c_ref[...], w1c_ref[...],
                          preferred_element_type=jnp.float32)


def _pre_body(xw_o_ref, xw_c_ref, deg_ref, y_o_ref, y_c_ref):
  dis_o = lax.rsqrt(deg_ref[0, :, 0:1] + 1.0)
  dis_c = lax.rsqrt(deg_ref[1, :, 0:1] + 1.0)
  y_o_ref[...] = xw_o_ref[...] * dis_o
  y_c_ref[...] = xw_c_ref[...] * dis_c


def _mid_body(acc_o_ref, acc_c_ref, y_o_ref, y_c_ref, deg_ref,
              b1o_ref, b1c_ref, w2o_ref, w2c_ref, y2o_ref, y2c_ref):
  dis_o = lax.rsqrt(deg_ref[0, :, 0:1] + 1.0)
  dis_c = lax.rsqrt(deg_ref[1, :, 0:1] + 1.0)
  h_o = jnp.maximum(dis_o * (acc_o_ref[...] + y_o_ref[...]) + b1o_ref[...],
                    0.0)
  h_c = jnp.maximum(dis_c * (acc_c_ref[...] + y_c_ref[...]) + b1c_ref[...],
                    0.0)
  y2o_ref[...] = jnp.dot(h_o, w2o_ref[...],
                         preferred_element_type=jnp.float32) * dis_o
  y2c_ref[...] = jnp.dot(h_c, w2c_ref[...],
                         preferred_element_type=jnp.float32) * dis_c


def _fin_body(acc_o_ref, acc_c_ref, y2o_ref, y2c_ref, deg_ref,
              b2o_ref, b2c_ref, batch_ref, wl_ref, bl_ref,
              x2o_ref, x2c_ref, hout_ref, pool_acc):
  i = pl.program_id(0)
  dis_o = lax.rsqrt(deg_ref[0, :, 0:1] + 1.0)
  dis_c = lax.rsqrt(deg_ref[1, :, 0:1] + 1.0)
  x2o = dis_o * (acc_o_ref[...] + y2o_ref[...]) + b2o_ref[...]
  x2c = dis_c * (acc_c_ref[...] + y2c_ref[...]) + b2c_ref[...]
  x2o_ref[...] = x2o
  x2c_ref[...] = x2c
  gi = lax.broadcasted_iota(jnp.int32, (G, R), 0).astype(jnp.float32)
  mt = jnp.where(batch_ref[0] == gi, 1.0, 0.0)
  contrib = jnp.dot(mt, x2c, preferred_element_type=jnp.float32)

  @pl.when(i == 0)
  def _():
    pool_acc[...] = contrib

  @pl.when(i > 0)
  def _():
    pool_acc[...] = pool_acc[...] + contrib

  @pl.when(i == NB - 1)
  def _():
    hout_ref[...] = jnp.dot(pool_acc[...], wl_ref[...],
                            preferred_element_type=jnp.float32) + bl_ref[...]


def _row_spec():
  return pl.BlockSpec((R, H), lambda i: (i, 0))


def _deg_spec():
  return pl.BlockSpec((NC, R, 16), lambda i: (0, i, 0))


def _full_spec():
  return pl.BlockSpec((D, H), lambda i: (0, 0))


def _bias_spec():
  return pl.BlockSpec((1, H), lambda i: (0, 0))


_xw_call = pl.pallas_call(
    _xw_body,
    grid=(NB,),
    in_specs=[_row_spec(), _row_spec(), _full_spec(), _full_spec()],
    out_specs=(_row_spec(), _row_spec()),
    out_shape=(jax.ShapeDtypeStruct((NP, H), jnp.float32),
               jax.ShapeDtypeStruct((NP, H), jnp.float32)),
)

_pre_call = pl.pallas_call(
    _pre_body,
    grid=(NB,),
    in_specs=[_row_spec(), _row_spec(), _deg_spec()],
    out_specs=(_row_spec(), _row_spec()),
    out_shape=(jax.ShapeDtypeStruct((NP, H), jnp.float32),
               jax.ShapeDtypeStruct((NP, H), jnp.float32)),
)

_mid_call = pl.pallas_call(
    _mid_body,
    grid=(NB,),
    in_specs=[_row_spec(), _row_spec(), _row_spec(), _row_spec(), _deg_spec(),
              _bias_spec(), _bias_spec(), _full_spec(), _full_spec()],
    out_specs=(_row_spec(), _row_spec()),
    out_shape=(jax.ShapeDtypeStruct((NP, H), jnp.float32),
               jax.ShapeDtypeStruct((NP, H), jnp.float32)),
)

_fin_call = pl.pallas_call(
    _fin_body,
    grid=(NB,),
    in_specs=[_row_spec(), _row_spec(), _row_spec(), _row_spec(), _deg_spec(),
              _bias_spec(), _bias_spec(),
              pl.BlockSpec((1, 1, R), lambda i: (i, 0, 0)),
              _full_spec(), _bias_spec()],
    out_specs=(_row_spec(), _row_spec(),
               pl.BlockSpec((G, H), lambda i: (0, 0))),
    out_shape=(jax.ShapeDtypeStruct((NP, H), jnp.float32),
               jax.ShapeDtypeStruct((NP, H), jnp.float32),
               jax.ShapeDtypeStruct((G, H), jnp.float32)),
    scratch_shapes=[pltpu.VMEM((G, H), jnp.float32)],
)


def _pad_edges(ei):
  src = ei[0].astype(jnp.int32)
  dst = ei[1].astype(jnp.int32)
  pad = jnp.full((EP - E,), N, dtype=jnp.int32)
  src = jnp.concatenate([src, pad]).reshape(NS, K, LB)
  dst = jnp.concatenate([dst, pad]).reshape(NS, K, LB)
  return src, dst


@jax.jit
def kernel(x_o, x_c, edge_index_o, edge_index_c, batch_o,
           W1o, b1o, W2o, b2o, W1c, b1c, W2c, b2c, Wl1, bl1):
  f32 = jnp.float32
  src_o, dst_o = _pad_edges(edge_index_o)
  src_c, dst_c = _pad_edges(edge_index_c)
  src_all = jnp.stack([src_o, src_c])
  dst_all = jnp.stack([dst_o, dst_c])
  src_all_p = src_all.reshape(NC, NS, KP, LBP)
  dst_all_p = dst_all.reshape(NC, NS, KP, LBP)

  xpad = jnp.zeros((NP - N, D), f32)
  x_o_p = jnp.concatenate([x_o.astype(f32), xpad])
  x_c_p = jnp.concatenate([x_c.astype(f32), xpad])

  ones_hbm = jnp.concatenate(
      [jnp.ones((LB, 1), f32), jnp.zeros((LB, 15), f32)], axis=1)
  z16 = jnp.zeros((RPS, 16), f32)
  z128 = jnp.zeros((RPS, H), f32)

  deg_all = _deg_kernel(dst_all, ones_hbm, z16)
  xw_o, xw_c = _xw_call(x_o_p, x_c_p, W1o.astype(f32), W1c.astype(f32))

  y1o, y1c = _pre_call(xw_o, xw_c, deg_all)
  acc1o, acc1c = _prop_kernel(src_all_p, dst_all_p, y1o, y1c, z128)
  y2o, y2c = _mid_call(acc1o, acc1c, y1o, y1c, deg_all,
                       b1o.reshape(1, H).astype(f32),
                       b1c.reshape(1, H).astype(f32),
                       W2o.astype(f32), W2c.astype(f32))
  acc2o, acc2c = _prop_kernel(src_all_p, dst_all_p, y2o, y2c, z128)

  batch_p = jnp.concatenate(
      [batch_o.astype(jnp.int32), jnp.full((NP - N,), G, jnp.int32)])
  batch_f = batch_p.astype(f32).reshape(NB, 1, R)

  x2o, x2c, hout = _fin_call(acc2o, acc2c, y2o, y2c, deg_all,
                             b2o.reshape(1, H).astype(f32),
                             b2c.reshape(1, H).astype(f32),
                             batch_f, Wl1.astype(f32),
                             bl1.reshape(1, H).astype(f32))
  return (hout, x2o[:N], x2c[:N])


# speedup vs baseline: 1.0925x; 1.0089x over previous
import functools

import jax
import jax.numpy as jnp
from jax import lax
from jax.experimental import pallas as pl
from jax.experimental.pallas import tpu as pltpu
from jax.experimental.pallas import tpu_sc as plsc

N = 10000
E = 320000
D = 128
H = 128
G = 128

NC = 2
NS = 16
LB = 128
K = 160
GR = 8
EP = NS * K * LB
LBP = 64
KP = EP // (NS * LBP)
GRP = 40
ND = 4
NP = 10240
RPS = NP // NS
R = 512
NB = NP // R

_mesh = plsc.VectorSubcoreMesh(
    core_axis_name="c", subcore_axis_name="s", num_cores=NC, num_subcores=NS)


def _deg_body(dst_all, ones_hbm, z_hbm, deg_out, dst_v, ones_v, sh_deg,
              sem_a, sem_b):
  cid = lax.axis_index("c")
  sid = lax.axis_index("s")
  pltpu.sync_copy(dst_all.at[cid, sid], dst_v)
  pltpu.sync_copy(ones_hbm, ones_v)
  pltpu.sync_copy(z_hbm, sh_deg.at[pl.ds(sid * RPS, RPS)])
  plsc.subcore_barrier()

  def body(g, carry):
    ds = []
    for j in range(16):
      sem = sem_a if j < 8 else sem_b
      ds.append(pltpu.async_copy(ones_v, sh_deg.at[dst_v.at[g * 16 + j]],
                                 sem, add=True))
    for d in ds:
      d.wait()
    return carry

  lax.fori_loop(0, K // 16, body, 0)
  plsc.subcore_barrier()
  pltpu.sync_copy(sh_deg.at[pl.ds(sid * RPS, RPS)],
                  deg_out.at[cid, pl.ds(sid * RPS, RPS)])


_deg_kernel = pl.kernel(
    _deg_body,
    out_type=jax.ShapeDtypeStruct((NC, NP, 16), jnp.float32),
    mesh=_mesh,
    scratch_types=[
        pltpu.VMEM((K, LB), jnp.int32),
        pltpu.VMEM((LB, 16), jnp.float32),
        pltpu.VMEM_SHARED((NP, 16), jnp.float32),
        pltpu.SemaphoreType.DMA,
        pltpu.SemaphoreType.DMA,
    ],
)


def _prop_body(src_all, dst_all, y_o, y_c, z_hbm, acc_o, acc_c,
               src_v, dst_v, r0, r1, r2, r3,
               sh_acc, g0, g1, g2, g3, s0, s1, s2, s3):
  cid = lax.axis_index("c")
  sid = lax.axis_index("s")
  pltpu.sync_copy(z_hbm, sh_acc.at[pl.ds(sid * RPS, RPS)])
  plsc.subcore_barrier()

  rows = (r0, r1, r2, r3)
  gsem = (g0, g1, g2, g3)
  ssem = (s0, s1, s2, s3)

  def run_branch(y_ref, acc_ref):
    def body(g, carry):
      pltpu.sync_copy(src_all.at[cid, sid, pl.ds(g * GRP, GRP)], src_v)
      pltpu.sync_copy(dst_all.at[cid, sid, pl.ds(g * GRP, GRP)], dst_v)
      gd = [None] * ND
      sd = [None] * ND
      for j in range(ND - 1):
        gd[j] = pltpu.async_copy(y_ref.at[src_v.at[j]], rows[j], gsem[j])
      for j in range(GRP):
        p = j % ND
        gd[p].wait()
        nj = j + ND - 1
        if nj < GRP:
          q = nj % ND
          if sd[q] is not None:
            sd[q].wait()
          gd[q] = pltpu.async_copy(y_ref.at[src_v.at[nj]], rows[q], gsem[q])
        sd[p] = pltpu.async_copy(rows[p], sh_acc.at[dst_v.at[j]], ssem[p],
                                 add=True)
      for j in range(GRP - ND, GRP):
        sd[j % ND].wait()
      return carry

    lax.fori_loop(0, KP // GRP, body, 0)
    plsc.subcore_barrier()
    pltpu.sync_copy(sh_acc.at[pl.ds(sid * RPS, RPS)],
                    acc_ref.at[pl.ds(sid * RPS, RPS)])

  @pl.when(cid == 0)
  def _():
    run_branch(y_o, acc_o)

  @pl.when(cid == 1)
  def _():
    run_branch(y_c, acc_c)


_prop_kernel = pl.kernel(
    _prop_body,
    out_type=(jax.ShapeDtypeStruct((NP, H), jnp.float32),
              jax.ShapeDtypeStruct((NP, H), jnp.float32)),
    mesh=_mesh,
    scratch_types=[
        pltpu.VMEM((GRP, LBP), jnp.int32),
        pltpu.VMEM((GRP, LBP), jnp.int32),
        pltpu.VMEM((LBP, H), jnp.float32),
        pltpu.VMEM((LBP, H), jnp.float32),
        pltpu.VMEM((LBP, H), jnp.float32),
        pltpu.VMEM((LBP, H), jnp.float32),
        pltpu.VMEM_SHARED((NP, H), jnp.float32),
    ] + [pltpu.SemaphoreType.DMA] * 8,
)


def _xw_body(x_o_ref, x_c_ref, w1o_ref, w1c_ref, xw_o_ref, xw_c_ref):
  xw_o_ref[...] = jnp.dot(x_o_ref[...], w1o_ref[...],
                          preferred_element_type=jnp.float32)
  xw_c_ref[...] = jnp.dot(x_c_ref[...], w1c_ref[...],
                          preferred_element_type=jnp.float32)


def _pre_body(xw_o_ref, xw_c_ref, deg_ref, y_o_ref, y_c_ref):
  dis_o = lax.rsqrt(deg_ref[0, :, 0:1] + 1.0)
  dis_c = lax.rsqrt(deg_ref[1, :, 0:1] + 1.0)
  y_o_ref[...] = xw_o_ref[...] * dis_o
  y_c_ref[...] = xw_c_ref[...] * dis_c


def _mid_body(acc_o_ref, acc_c_ref, y_o_ref, y_c_ref, deg_ref,
              b1o_ref, b1c_ref, w2o_ref, w2c_ref, y2o_ref, y2c_ref):
  dis_o = lax.rsqrt(deg_ref[0, :, 0:1] + 1.0)
  dis_c = lax.rsqrt(deg_ref[1, :, 0:1] + 1.0)
  h_o = jnp.maximum(dis_o * (acc_o_ref[...] + y_o_ref[...]) + b1o_ref[...],
                    0.0)
  h_c = jnp.maximum(dis_c * (acc_c_ref[...] + y_c_ref[...]) + b1c_ref[...],
                    0.0)
  y2o_ref[...] = jnp.dot(h_o, w2o_ref[...],
                         preferred_element_type=jnp.float32) * dis_o
  y2c_ref[...] = jnp.dot(h_c, w2c_ref[...],
                         preferred_element_type=jnp.float32) * dis_c


def _fin_body(acc_o_ref, acc_c_ref, y2o_ref, y2c_ref, deg_ref,
              b2o_ref, b2c_ref, batch_ref, wl_ref, bl_ref,
              x2o_ref, x2c_ref, hout_ref, pool_acc):
  i = pl.program_id(0)
  dis_o = lax.rsqrt(deg_ref[0, :, 0:1] + 1.0)
  dis_c = lax.rsqrt(deg_ref[1, :, 0:1] + 1.0)
  x2o = dis_o * (acc_o_ref[...] + y2o_ref[...]) + b2o_ref[...]
  x2c = dis_c * (acc_c_ref[...] + y2c_ref[...]) + b2c_ref[...]
  x2o_ref[...] = x2o
  x2c_ref[...] = x2c
  gi = lax.broadcasted_iota(jnp.int32, (G, R), 0).astype(jnp.float32)
  mt = jnp.where(batch_ref[0] == gi, 1.0, 0.0)
  contrib = jnp.dot(mt, x2c, preferred_element_type=jnp.float32)

  @pl.when(i == 0)
  def _():
    pool_acc[...] = contrib

  @pl.when(i > 0)
  def _():
    pool_acc[...] = pool_acc[...] + contrib

  @pl.when(i == NB - 1)
  def _():
    hout_ref[...] = jnp.dot(pool_acc[...], wl_ref[...],
                            preferred_element_type=jnp.float32) + bl_ref[...]


def _row_spec():
  return pl.BlockSpec((R, H), lambda i: (i, 0))


def _deg_spec():
  return pl.BlockSpec((NC, R, 16), lambda i: (0, i, 0))


def _full_spec():
  return pl.BlockSpec((D, H), lambda i: (0, 0))


def _bias_spec():
  return pl.BlockSpec((1, H), lambda i: (0, 0))


_xw_call = pl.pallas_call(
    _xw_body,
    grid=(NB,),
    in_specs=[_row_spec(), _row_spec(), _full_spec(), _full_spec()],
    out_specs=(_row_spec(), _row_spec()),
    out_shape=(jax.ShapeDtypeStruct((NP, H), jnp.float32),
               jax.ShapeDtypeStruct((NP, H), jnp.float32)),
)

_pre_call = pl.pallas_call(
    _pre_body,
    grid=(NB,),
    in_specs=[_row_spec(), _row_spec(), _deg_spec()],
    out_specs=(_row_spec(), _row_spec()),
    out_shape=(jax.ShapeDtypeStruct((NP, H), jnp.float32),
               jax.ShapeDtypeStruct((NP, H), jnp.float32)),
)

_mid_call = pl.pallas_call(
    _mid_body,
    grid=(NB,),
    in_specs=[_row_spec(), _row_spec(), _row_spec(), _row_spec(), _deg_spec(),
              _bias_spec(), _bias_spec(), _full_spec(), _full_spec()],
    out_specs=(_row_spec(), _row_spec()),
    out_shape=(jax.ShapeDtypeStruct((NP, H), jnp.float32),
               jax.ShapeDtypeStruct((NP, H), jnp.float32)),
)

_fin_call = pl.pallas_call(
    _fin_body,
    grid=(NB,),
    in_specs=[_row_spec(), _row_spec(), _row_spec(), _row_spec(), _deg_spec(),
              _bias_spec(), _bias_spec(),
              pl.BlockSpec((1, 1, R), lambda i: (i, 0, 0)),
              _full_spec(), _bias_spec()],
    out_specs=(_row_spec(), _row_spec(),
               pl.BlockSpec((G, H), lambda i: (0, 0))),
    out_shape=(jax.ShapeDtypeStruct((NP, H), jnp.float32),
               jax.ShapeDtypeStruct((NP, H), jnp.float32),
               jax.ShapeDtypeStruct((G, H), jnp.float32)),
    scratch_shapes=[pltpu.VMEM((G, H), jnp.float32)],
)


def _pad_edges(ei):
  src = ei[0].astype(jnp.int32)
  dst = ei[1].astype(jnp.int32)
  pad = jnp.full((EP - E,), N, dtype=jnp.int32)
  src = jnp.concatenate([src, pad]).reshape(NS, K, LB)
  dst = jnp.concatenate([dst, pad]).reshape(NS, K, LB)
  return src, dst


@jax.jit
def kernel(x_o, x_c, edge_index_o, edge_index_c, batch_o,
           W1o, b1o, W2o, b2o, W1c, b1c, W2c, b2c, Wl1, bl1):
  f32 = jnp.float32
  src_o, dst_o = _pad_edges(edge_index_o)
  src_c, dst_c = _pad_edges(edge_index_c)
  src_all = jnp.stack([src_o, src_c])
  dst_all = jnp.stack([dst_o, dst_c])
  src_all_p = src_all.reshape(NC, NS, KP, LBP)
  dst_all_p = dst_all.reshape(NC, NS, KP, LBP)

  xpad = jnp.zeros((NP - N, D), f32)
  x_o_p = jnp.concatenate([x_o.astype(f32), xpad])
  x_c_p = jnp.concatenate([x_c.astype(f32), xpad])

  ones_hbm = jnp.concatenate(
      [jnp.ones((LB, 1), f32), jnp.zeros((LB, 15), f32)], axis=1)
  z16 = jnp.zeros((RPS, 16), f32)
  z128 = jnp.zeros((RPS, H), f32)

  deg_all = _deg_kernel(dst_all, ones_hbm, z16)
  xw_o, xw_c = _xw_call(x_o_p, x_c_p, W1o.astype(f32), W1c.astype(f32))

  y1o, y1c = _pre_call(xw_o, xw_c, deg_all)
  acc1o, acc1c = _prop_kernel(src_all_p, dst_all_p, y1o, y1c, z128)
  y2o, y2c = _mid_call(acc1o, acc1c, y1o, y1c, deg_all,
                       b1o.reshape(1, H).astype(f32),
                       b1c.reshape(1, H).astype(f32),
                       W2o.astype(f32), W2c.astype(f32))
  acc2o, acc2c = _prop_kernel(src_all_p, dst_all_p, y2o, y2c, z128)

  batch_p = jnp.concatenate(
      [batch_o.astype(jnp.int32), jnp.full((NP - N,), G, jnp.int32)])
  batch_f = batch_p.astype(f32).reshape(NB, 1, R)

  x2o, x2c, hout = _fin_call(acc2o, acc2c, y2o, y2c, deg_all,
                             b2o.reshape(1, H).astype(f32),
                             b2c.reshape(1, H).astype(f32),
                             batch_f, Wl1.astype(f32),
                             bl1.reshape(1, H).astype(f32))
  return (hout, x2o[:N], x2c[:N])


# speedup vs baseline: 1.1416x; 1.0449x over previous
import functools

import jax
import jax.numpy as jnp
from jax import lax
from jax.experimental import pallas as pl
from jax.experimental.pallas import tpu as pltpu
from jax.experimental.pallas import tpu_sc as plsc

N = 10000
E = 320000
D = 128
H = 128
G = 128

NC = 2
NS = 16
LB = 128
K = 160
GR = 8
EP = NS * K * LB
LBP = 64
KP = EP // (NS * LBP)
GRP = 64
ND = 4
NP = 10112
RPS = NP // NS
R = 632
NB = NP // R

_mesh = plsc.VectorSubcoreMesh(
    core_axis_name="c", subcore_axis_name="s", num_cores=NC, num_subcores=NS)


def _deg_body(dst_all, ones_hbm, z_hbm, deg_out, dst_v, ones_v, sh_deg,
              sem_a, sem_b):
  cid = lax.axis_index("c")
  sid = lax.axis_index("s")
  pltpu.sync_copy(dst_all.at[cid, sid], dst_v)
  pltpu.sync_copy(ones_hbm, ones_v)
  pltpu.sync_copy(z_hbm, sh_deg.at[pl.ds(sid * RPS, RPS)])
  plsc.subcore_barrier()

  def body(g, carry):
    ds = []
    for j in range(16):
      sem = sem_a if j < 8 else sem_b
      ds.append(pltpu.async_copy(ones_v, sh_deg.at[dst_v.at[g * 16 + j]],
                                 sem, add=True))
    for d in ds:
      d.wait()
    return carry

  lax.fori_loop(0, K // 16, body, 0)
  plsc.subcore_barrier()
  pltpu.sync_copy(sh_deg.at[pl.ds(sid * RPS, RPS)],
                  deg_out.at[cid, pl.ds(sid * RPS, RPS)])


_deg_kernel = pl.kernel(
    _deg_body,
    out_type=jax.ShapeDtypeStruct((NC, NP, 16), jnp.float32),
    mesh=_mesh,
    scratch_types=[
        pltpu.VMEM((K, LB), jnp.int32),
        pltpu.VMEM((LB, 16), jnp.float32),
        pltpu.VMEM_SHARED((NP, 16), jnp.float32),
        pltpu.SemaphoreType.DMA,
        pltpu.SemaphoreType.DMA,
    ],
)


def _prop_body(src_all, dst_all, y_o, y_c, z_hbm, acc_o, acc_c,
               src_v, dst_v, r0, r1, r2, r3,
               sh_acc, g0, g1, g2, g3, s0, s1, s2, s3):
  cid = lax.axis_index("c")
  sid = lax.axis_index("s")
  pltpu.sync_copy(z_hbm, sh_acc.at[pl.ds(sid * RPS, RPS)])
  plsc.subcore_barrier()

  rows = (r0, r1, r2, r3)
  gsem = (g0, g1, g2, g3)
  ssem = (s0, s1, s2, s3)

  def run_branch(y_ref, acc_ref):
    def body(g, carry):
      pltpu.sync_copy(src_all.at[cid, sid, pl.ds(g * GRP, GRP)], src_v)
      pltpu.sync_copy(dst_all.at[cid, sid, pl.ds(g * GRP, GRP)], dst_v)
      gd = [None] * ND
      sd = [None] * ND
      for j in range(ND - 1):
        gd[j] = pltpu.async_copy(y_ref.at[src_v.at[j]], rows[j], gsem[j])
      for j in range(GRP):
        p = j % ND
        gd[p].wait()
        nj = j + ND - 1
        if nj < GRP:
          q = nj % ND
          if sd[q] is not None:
            sd[q].wait()
          gd[q] = pltpu.async_copy(y_ref.at[src_v.at[nj]], rows[q], gsem[q])
        sd[p] = pltpu.async_copy(rows[p], sh_acc.at[dst_v.at[j]], ssem[p],
                                 add=True)
      for j in range(GRP - ND, GRP):
        sd[j % ND].wait()
      return carry

    lax.fori_loop(0, KP // GRP, body, 0)
    plsc.subcore_barrier()
    pltpu.sync_copy(sh_acc.at[pl.ds(sid * RPS, RPS)],
                    acc_ref.at[pl.ds(sid * RPS, RPS)])

  @pl.when(cid == 0)
  def _():
    run_branch(y_o, acc_o)

  @pl.when(cid == 1)
  def _():
    run_branch(y_c, acc_c)


_prop_kernel = pl.kernel(
    _prop_body,
    out_type=(jax.ShapeDtypeStruct((NP, H), jnp.float32),
              jax.ShapeDtypeStruct((NP, H), jnp.float32)),
    mesh=_mesh,
    scratch_types=[
        pltpu.VMEM((GRP, LBP), jnp.int32),
        pltpu.VMEM((GRP, LBP), jnp.int32),
        pltpu.VMEM((LBP, H), jnp.float32),
        pltpu.VMEM((LBP, H), jnp.float32),
        pltpu.VMEM((LBP, H), jnp.float32),
        pltpu.VMEM((LBP, H), jnp.float32),
        pltpu.VMEM_SHARED((NP, H), jnp.float32),
    ] + [pltpu.SemaphoreType.DMA] * 8,
)


def _xw_body(x_o_ref, x_c_ref, w1o_ref, w1c_ref, xw_o_ref, xw_c_ref):
  xw_o_ref[...] = jnp.dot(x_o_ref[...], w1o_ref[...],
                          preferred_element_type=jnp.float32)
  xw_c_ref[...] = jnp.dot(x_c_ref[...], w1c_ref[...],
                          preferred_element_type=jnp.float32)


def _pre_body(xw_o_ref, xw_c_ref, deg_ref, y_o_ref, y_c_ref):
  dis_o = lax.rsqrt(deg_ref[0, :, 0:1] + 1.0)
  dis_c = lax.rsqrt(deg_ref[1, :, 0:1] + 1.0)
  y_o_ref[...] = xw_o_ref[...] * dis_o
  y_c_ref[...] = xw_c_ref[...] * dis_c


def _mid_body(acc_o_ref, acc_c_ref, y_o_ref, y_c_ref, deg_ref,
              b1o_ref, b1c_ref, w2o_ref, w2c_ref, y2o_ref, y2c_ref):
  dis_o = lax.rsqrt(deg_ref[0, :, 0:1] + 1.0)
  dis_c = lax.rsqrt(deg_ref[1, :, 0:1] + 1.0)
  h_o = jnp.maximum(dis_o * (acc_o_ref[...] + y_o_ref[...]) + b1o_ref[...],
                    0.0)
  h_c = jnp.maximum(dis_c * (acc_c_ref[...] + y_c_ref[...]) + b1c_ref[...],
                    0.0)
  y2o_ref[...] = jnp.dot(h_o, w2o_ref[...],
                         preferred_element_type=jnp.float32) * dis_o
  y2c_ref[...] = jnp.dot(h_c, w2c_ref[...],
                         preferred_element_type=jnp.float32) * dis_c


def _fin_body(acc_o_ref, acc_c_ref, y2o_ref, y2c_ref, deg_ref,
              b2o_ref, b2c_ref, batch_ref, wl_ref, bl_ref,
              x2o_ref, x2c_ref, hout_ref, pool_acc):
  i = pl.program_id(0)
  dis_o = lax.rsqrt(deg_ref[0, :, 0:1] + 1.0)
  dis_c = lax.rsqrt(deg_ref[1, :, 0:1] + 1.0)
  x2o = dis_o * (acc_o_ref[...] + y2o_ref[...]) + b2o_ref[...]
  x2c = dis_c * (acc_c_ref[...] + y2c_ref[...]) + b2c_ref[...]
  x2o_ref[...] = x2o
  x2c_ref[...] = x2c
  gi = lax.broadcasted_iota(jnp.int32, (G, R), 0).astype(jnp.float32)
  mt = jnp.where(batch_ref[0] == gi, 1.0, 0.0)
  contrib = jnp.dot(mt, x2c, preferred_element_type=jnp.float32)

  @pl.when(i == 0)
  def _():
    pool_acc[...] = contrib

  @pl.when(i > 0)
  def _():
    pool_acc[...] = pool_acc[...] + contrib

  @pl.when(i == NB - 1)
  def _():
    hout_ref[...] = jnp.dot(pool_acc[...], wl_ref[...],
                            preferred_element_type=jnp.float32) + bl_ref[...]


def _row_spec():
  return pl.BlockSpec((R, H), lambda i: (i, 0))


def _deg_spec():
  return pl.BlockSpec((NC, R, 16), lambda i: (0, i, 0))


def _full_spec():
  return pl.BlockSpec((D, H), lambda i: (0, 0))


def _bias_spec():
  return pl.BlockSpec((1, H), lambda i: (0, 0))


_xw_call = pl.pallas_call(
    _xw_body,
    grid=(NB,),
    in_specs=[_row_spec(), _row_spec(), _full_spec(), _full_spec()],
    out_specs=(_row_spec(), _row_spec()),
    out_shape=(jax.ShapeDtypeStruct((NP, H), jnp.float32),
               jax.ShapeDtypeStruct((NP, H), jnp.float32)),
)

_pre_call = pl.pallas_call(
    _pre_body,
    grid=(NB,),
    in_specs=[_row_spec(), _row_spec(), _deg_spec()],
    out_specs=(_row_spec(), _row_spec()),
    out_shape=(jax.ShapeDtypeStruct((NP, H), jnp.float32),
               jax.ShapeDtypeStruct((NP, H), jnp.float32)),
)

_mid_call = pl.pallas_call(
    _mid_body,
    grid=(NB,),
    in_specs=[_row_spec(), _row_spec(), _row_spec(), _row_spec(), _deg_spec(),
              _bias_spec(), _bias_spec(), _full_spec(), _full_spec()],
    out_specs=(_row_spec(), _row_spec()),
    out_shape=(jax.ShapeDtypeStruct((NP, H), jnp.float32),
               jax.ShapeDtypeStruct((NP, H), jnp.float32)),
)

_fin_call = pl.pallas_call(
    _fin_body,
    grid=(NB,),
    in_specs=[_row_spec(), _row_spec(), _row_spec(), _row_spec(), _deg_spec(),
              _bias_spec(), _bias_spec(),
              pl.BlockSpec((1, 1, R), lambda i: (i, 0, 0)),
              _full_spec(), _bias_spec()],
    out_specs=(_row_spec(), _row_spec(),
               pl.BlockSpec((G, H), lambda i: (0, 0))),
    out_shape=(jax.ShapeDtypeStruct((NP, H), jnp.float32),
               jax.ShapeDtypeStruct((NP, H), jnp.float32),
               jax.ShapeDtypeStruct((G, H), jnp.float32)),
    scratch_shapes=[pltpu.VMEM((G, H), jnp.float32)],
)


def _pad_edges(ei):
  src = ei[0].astype(jnp.int32)
  dst = ei[1].astype(jnp.int32)
  pad = jnp.full((EP - E,), N, dtype=jnp.int32)
  src = jnp.concatenate([src, pad]).reshape(NS, K, LB)
  dst = jnp.concatenate([dst, pad]).reshape(NS, K, LB)
  return src, dst


@jax.jit
def kernel(x_o, x_c, edge_index_o, edge_index_c, batch_o,
           W1o, b1o, W2o, b2o, W1c, b1c, W2c, b2c, Wl1, bl1):
  f32 = jnp.float32
  src_o, dst_o = _pad_edges(edge_index_o)
  src_c, dst_c = _pad_edges(edge_index_c)
  src_all = jnp.stack([src_o, src_c])
  dst_all = jnp.stack([dst_o, dst_c])
  src_all_p = src_all.reshape(NC, NS, KP, LBP)
  dst_all_p = dst_all.reshape(NC, NS, KP, LBP)

  xpad = jnp.zeros((NP - N, D), f32)
  x_o_p = jnp.concatenate([x_o.astype(f32), xpad])
  x_c_p = jnp.concatenate([x_c.astype(f32), xpad])

  ones_hbm = jnp.concatenate(
      [jnp.ones((LB, 1), f32), jnp.zeros((LB, 15), f32)], axis=1)
  z16 = jnp.zeros((RPS, 16), f32)
  z128 = jnp.zeros((RPS, H), f32)

  deg_all = _deg_kernel(dst_all, ones_hbm, z16)
  xw_o, xw_c = _xw_call(x_o_p, x_c_p, W1o.astype(f32), W1c.astype(f32))

  y1o, y1c = _pre_call(xw_o, xw_c, deg_all)
  acc1o, acc1c = _prop_kernel(src_all_p, dst_all_p, y1o, y1c, z128)
  y2o, y2c = _mid_call(acc1o, acc1c, y1o, y1c, deg_all,
                       b1o.reshape(1, H).astype(f32),
                       b1c.reshape(1, H).astype(f32),
                       W2o.astype(f32), W2c.astype(f32))
  acc2o, acc2c = _prop_kernel(src_all_p, dst_all_p, y2o, y2c, z128)

  batch_p = jnp.concatenate(
      [batch_o.astype(jnp.int32), jnp.full((NP - N,), G, jnp.int32)])
  batch_f = batch_p.astype(f32).reshape(NB, 1, R)

  x2o, x2c, hout = _fin_call(acc2o, acc2c, y2o, y2c, deg_all,
                             b2o.reshape(1, H).astype(f32),
                             b2c.reshape(1, H).astype(f32),
                             batch_f, Wl1.astype(f32),
                             bl1.reshape(1, H).astype(f32))
  return (hout, x2o[:N], x2c[:N])


# speedup vs baseline: 1.1418x; 1.0002x over previous
import jax
import jax.numpy as jnp
from jax import lax
from jax.experimental import pallas as pl
from jax.experimental.pallas import tpu as pltpu
from jax.experimental.pallas import tpu_sc as plsc

N = 10000
E = 320000
D = 128
H = 128
G = 128

NC = 2
NS = 16
LB = 128
K = 160
GR = 8
EP = NS * K * LB
LBP = 64
KP = EP // (NS * LBP)
GRP = 64
ND = 4
NP = 10112
RPS = NP // NS
R = 632
NB = NP // R

_mesh = plsc.VectorSubcoreMesh(
    core_axis_name="c", subcore_axis_name="s", num_cores=NC, num_subcores=NS)


def _deg_body(dst_all, ones_hbm, z_hbm, deg_out, dst_v, ones_v, sh_deg,
              sem_a, sem_b):
  cid = lax.axis_index("c")
  sid = lax.axis_index("s")
  pltpu.sync_copy(dst_all.at[cid, sid], dst_v)
  pltpu.sync_copy(ones_hbm, ones_v)
  pltpu.sync_copy(z_hbm, sh_deg.at[pl.ds(sid * RPS, RPS)])
  plsc.subcore_barrier()

  def body(g, carry):
    ds = []
    for j in range(16):
      sem = sem_a if j < 8 else sem_b
      ds.append(pltpu.async_copy(ones_v, sh_deg.at[dst_v.at[g * 16 + j]],
                                 sem, add=True))
    for d in ds:
      d.wait()
    return carry

  lax.fori_loop(0, K // 16, body, 0)
  plsc.subcore_barrier()
  pltpu.sync_copy(sh_deg.at[pl.ds(sid * RPS, RPS)],
                  deg_out.at[cid, pl.ds(sid * RPS, RPS)])


_deg_kernel = pl.kernel(
    _deg_body,
    out_type=jax.ShapeDtypeStruct((NC, NP, 16), jnp.float32),
    mesh=_mesh,
    scratch_types=[
        pltpu.VMEM((K, LB), jnp.int32),
        pltpu.VMEM((LB, 16), jnp.float32),
        pltpu.VMEM_SHARED((NP, 16), jnp.float32),
        pltpu.SemaphoreType.DMA,
        pltpu.SemaphoreType.DMA,
    ],
)


def _prop_body(src_all, dst_all, y_o, y_c, z_hbm, acc_o, acc_c,
               src_v, dst_v, r0, r1, r2, r3,
               sh_acc, g0, g1, g2, g3, s0, s1, s2, s3):
  cid = lax.axis_index("c")
  sid = lax.axis_index("s")
  pltpu.sync_copy(z_hbm, sh_acc.at[pl.ds(sid * RPS, RPS)])
  plsc.subcore_barrier()

  rows = (r0, r1, r2, r3)
  gsem = (g0, g1, g2, g3)
  ssem = (s0, s1, s2, s3)

  def run_branch(y_ref, acc_ref):
    def body(g, carry):
      pltpu.sync_copy(src_all.at[cid, sid, pl.ds(g * GRP, GRP)], src_v)
      pltpu.sync_copy(dst_all.at[cid, sid, pl.ds(g * GRP, GRP)], dst_v)
      gd = [None] * ND
      sd = [None] * ND
      for j in range(ND - 1):
        gd[j] = pltpu.async_copy(y_ref.at[src_v.at[j]], rows[j], gsem[j])
      for j in range(GRP):
        p = j % ND
        gd[p].wait()
        nj = j + ND - 1
        if nj < GRP:
          q = nj % ND
          if sd[q] is not None:
            sd[q].wait()
          gd[q] = pltpu.async_copy(y_ref.at[src_v.at[nj]], rows[q], gsem[q])
        sd[p] = pltpu.async_copy(rows[p], sh_acc.at[dst_v.at[j]], ssem[p],
                                 add=True)
      for j in range(GRP - ND, GRP):
        sd[j % ND].wait()
      return carry

    lax.fori_loop(0, KP // GRP, body, 0)
    plsc.subcore_barrier()
    pltpu.sync_copy(sh_acc.at[pl.ds(sid * RPS, RPS)],
                    acc_ref.at[pl.ds(sid * RPS, RPS)])

  @pl.when(cid == 0)
  def _():
    run_branch(y_o, acc_o)

  @pl.when(cid == 1)
  def _():
    run_branch(y_c, acc_c)


_prop_kernel = pl.kernel(
    _prop_body,
    out_type=(jax.ShapeDtypeStruct((NP, H), jnp.float32),
              jax.ShapeDtypeStruct((NP, H), jnp.float32)),
    mesh=_mesh,
    scratch_types=[
        pltpu.VMEM((GRP, LBP), jnp.int32),
        pltpu.VMEM((GRP, LBP), jnp.int32),
        pltpu.VMEM((LBP, H), jnp.float32),
        pltpu.VMEM((LBP, H), jnp.float32),
        pltpu.VMEM((LBP, H), jnp.float32),
        pltpu.VMEM((LBP, H), jnp.float32),
        pltpu.VMEM_SHARED((NP, H), jnp.float32),
    ] + [pltpu.SemaphoreType.DMA] * 8,
)


def _xw_body(x_o_ref, x_c_ref, w1o_ref, w1c_ref, xw_o_ref, xw_c_ref):
  xw_o_ref[...] = jnp.dot(x_o_ref[...], w1o_ref[...],
                          preferred_element_type=jnp.float32)
  xw_c_ref[...] = jnp.dot(x_c_ref[...], w1c_ref[...],
                          preferred_element_type=jnp.float32)


def _pre_body(xw_o_ref, xw_c_ref, deg_ref, y_o_ref, y_c_ref):
  dis_o = lax.rsqrt(deg_ref[0, :, 0:1] + 1.0)
  dis_c = lax.rsqrt(deg_ref[1, :, 0:1] + 1.0)
  y_o_ref[...] = xw_o_ref[...] * dis_o
  y_c_ref[...] = xw_c_ref[...] * dis_c


def _mid_body(acc_o_ref, acc_c_ref, y_o_ref, y_c_ref, deg_ref,
              b1o_ref, b1c_ref, w2o_ref, w2c_ref, y2o_ref, y2c_ref):
  dis_o = lax.rsqrt(deg_ref[0, :, 0:1] + 1.0)
  dis_c = lax.rsqrt(deg_ref[1, :, 0:1] + 1.0)
  h_o = jnp.maximum(dis_o * (acc_o_ref[...] + y_o_ref[...]) + b1o_ref[...],
                    0.0)
  h_c = jnp.maximum(dis_c * (acc_c_ref[...] + y_c_ref[...]) + b1c_ref[...],
                    0.0)
  y2o_ref[...] = jnp.dot(h_o, w2o_ref[...],
                         preferred_element_type=jnp.float32) * dis_o
  y2c_ref[...] = jnp.dot(h_c, w2c_ref[...],
                         preferred_element_type=jnp.float32) * dis_c


def _fin_body(acc_o_ref, acc_c_ref, y2o_ref, y2c_ref, deg_ref,
              b2o_ref, b2c_ref, batch_ref, wl_ref, bl_ref,
              x2o_ref, x2c_ref, hout_ref, pool_acc):
  i = pl.program_id(0)
  dis_o = lax.rsqrt(deg_ref[0, :, 0:1] + 1.0)
  dis_c = lax.rsqrt(deg_ref[1, :, 0:1] + 1.0)
  x2o = dis_o * (acc_o_ref[...] + y2o_ref[...]) + b2o_ref[...]
  x2c = dis_c * (acc_c_ref[...] + y2c_ref[...]) + b2c_ref[...]
  x2o_ref[...] = x2o
  x2c_ref[...] = x2c
  gi = lax.broadcasted_iota(jnp.int32, (G, R), 0).astype(jnp.float32)
  mt = jnp.where(batch_ref[0] == gi, 1.0, 0.0)
  contrib = jnp.dot(mt, x2c, preferred_element_type=jnp.float32)

  @pl.when(i == 0)
  def _():
    pool_acc[...] = contrib

  @pl.when(i > 0)
  def _():
    pool_acc[...] = pool_acc[...] + contrib

  @pl.when(i == NB - 1)
  def _():
    hout_ref[...] = jnp.dot(pool_acc[...], wl_ref[...],
                            preferred_element_type=jnp.float32) + bl_ref[...]


def _row_spec():
  return pl.BlockSpec((R, H), lambda i: (i, 0))


def _deg_spec():
  return pl.BlockSpec((NC, R, 16), lambda i: (0, i, 0))


def _full_spec():
  return pl.BlockSpec((D, H), lambda i: (0, 0))


def _bias_spec():
  return pl.BlockSpec((1, H), lambda i: (0, 0))


_xw_call = pl.pallas_call(
    _xw_body,
    grid=(NB,),
    in_specs=[_row_spec(), _row_spec(), _full_spec(), _full_spec()],
    out_specs=(_row_spec(), _row_spec()),
    out_shape=(jax.ShapeDtypeStruct((NP, H), jnp.float32),
               jax.ShapeDtypeStruct((NP, H), jnp.float32)),
)

_pre_call = pl.pallas_call(
    _pre_body,
    grid=(NB,),
    in_specs=[_row_spec(), _row_spec(), _deg_spec()],
    out_specs=(_row_spec(), _row_spec()),
    out_shape=(jax.ShapeDtypeStruct((NP, H), jnp.float32),
               jax.ShapeDtypeStruct((NP, H), jnp.float32)),
)

_mid_call = pl.pallas_call(
    _mid_body,
    grid=(NB,),
    in_specs=[_row_spec(), _row_spec(), _row_spec(), _row_spec(), _deg_spec(),
              _bias_spec(), _bias_spec(), _full_spec(), _full_spec()],
    out_specs=(_row_spec(), _row_spec()),
    out_shape=(jax.ShapeDtypeStruct((NP, H), jnp.float32),
               jax.ShapeDtypeStruct((NP, H), jnp.float32)),
)

_fin_call = pl.pallas_call(
    _fin_body,
    grid=(NB,),
    in_specs=[_row_spec(), _row_spec(), _row_spec(), _row_spec(), _deg_spec(),
              _bias_spec(), _bias_spec(),
              pl.BlockSpec((1, 1, R), lambda i: (i, 0, 0)),
              _full_spec(), _bias_spec()],
    out_specs=(_row_spec(), _row_spec(),
               pl.BlockSpec((G, H), lambda i: (0, 0))),
    out_shape=(jax.ShapeDtypeStruct((NP, H), jnp.float32),
               jax.ShapeDtypeStruct((NP, H), jnp.float32),
               jax.ShapeDtypeStruct((G, H), jnp.float32)),
    scratch_shapes=[pltpu.VMEM((G, H), jnp.float32)],
)


def _pad_edges(ei):
  src = ei[0].astype(jnp.int32)
  dst = ei[1].astype(jnp.int32)
  pad = jnp.full((EP - E,), N, dtype=jnp.int32)
  src = jnp.concatenate([src, pad]).reshape(NS, K, LB)
  dst = jnp.concatenate([dst, pad]).reshape(NS, K, LB)
  return src, dst


@jax.jit
def kernel(x_o, x_c, edge_index_o, edge_index_c, batch_o,
           W1o, b1o, W2o, b2o, W1c, b1c, W2c, b2c, Wl1, bl1):
  f32 = jnp.float32
  src_o, dst_o = _pad_edges(edge_index_o)
  src_c, dst_c = _pad_edges(edge_index_c)
  src_all = jnp.stack([src_o, src_c])
  dst_all = jnp.stack([dst_o, dst_c])
  src_all_p = src_all.reshape(NC, NS, KP, LBP)
  dst_all_p = dst_all.reshape(NC, NS, KP, LBP)

  xpad = jnp.zeros((NP - N, D), f32)
  x_o_p = jnp.concatenate([x_o.astype(f32), xpad])
  x_c_p = jnp.concatenate([x_c.astype(f32), xpad])

  ones_hbm = jnp.concatenate(
      [jnp.ones((LB, 1), f32), jnp.zeros((LB, 15), f32)], axis=1)
  z16 = jnp.zeros((RPS, 16), f32)
  z128 = jnp.zeros((RPS, H), f32)

  deg_all = _deg_kernel(dst_all, ones_hbm, z16)
  xw_o, xw_c = _xw_call(x_o_p, x_c_p, W1o.astype(f32), W1c.astype(f32))

  y1o, y1c = _pre_call(xw_o, xw_c, deg_all)
  acc1o, acc1c = _prop_kernel(src_all_p, dst_all_p, y1o, y1c, z128)
  y2o, y2c = _mid_call(acc1o, acc1c, y1o, y1c, deg_all,
                       b1o.reshape(1, H).astype(f32),
                       b1c.reshape(1, H).astype(f32),
                       W2o.astype(f32), W2c.astype(f32))
  acc2o, acc2c = _prop_kernel(src_all_p, dst_all_p, y2o, y2c, z128)

  batch_p = jnp.concatenate(
      [batch_o.astype(jnp.int32), jnp.full((NP - N,), G, jnp.int32)])
  batch_f = batch_p.astype(f32).reshape(NB, 1, R)

  x2o, x2c, hout = _fin_call(acc2o, acc2c, y2o, y2c, deg_all,
                             b2o.reshape(1, H).astype(f32),
                             b2c.reshape(1, H).astype(f32),
                             batch_f, Wl1.astype(f32),
                             bl1.reshape(1, H).astype(f32))
  return (hout, x2o[:N], x2c[:N])
